# Initial kernel scaffold; baseline (speedup 1.0000x reference)
import jax
import jax.numpy as jnp
from jax.experimental import pallas as pl

_N = 1000
_NP = 1024
_D = 64
_IN = 16
_K = 20
_B = 32
_BLK = 128
_BLKR = 256
_NR = _NP // _BLKR
_CNT = float(_B * _N)
_EPS = 1e-5


def _mask_kernel(wf_ref, wb_ref, bias_ref):
    r = pl.program_id(0)
    w = wf_ref[...]
    wb = wb_ref[...]
    nrm2 = jnp.maximum(jnp.sum(w * w, axis=1), 1e-12)
    n_full = jnp.sqrt(nrm2)
    nb2 = jnp.maximum(jnp.sum(wb * wb, axis=1), 1e-12)
    n_blk = jnp.sqrt(nb2)
    cos = jax.lax.dot_general(
        wb, w, (((1,), (1,)), ((), ())),
        preferred_element_type=jnp.float32)
    cos = cos / (n_blk[:, None] * n_full[None, :])
    col = jax.lax.broadcasted_iota(jnp.int32, (_BLK, _NP), 1)
    rowg = r * _BLK + jax.lax.broadcasted_iota(jnp.int32, (_BLK, _NP), 0)
    cur = jnp.where(col < _N, cos, -1e9)
    selected = col == rowg
    for _ in range(_K):
        m = jnp.max(cur, axis=1, keepdims=True)
        eq = cur == m
        idx = jnp.min(jnp.where(eq, col, _NP), axis=1, keepdims=True)
        sel = col == idx
        selected = jnp.logical_or(selected, sel)
        cur = jnp.where(sel, -2e9, cur)
    bias_ref[...] = jnp.where(selected, 0.0, -1e30).astype(jnp.float32)


def _attn_kernel(data_ref, lw_ref, emb_ref, atti_ref, attj_ref, attemi_ref,
                 attemj_ref, gb_ref, bias_ref, out_ref, s1_ref, s2_ref):
    r = pl.program_id(1)
    db = data_ref[0]
    xl = jax.lax.dot_general(
        db, lw_ref[...], (((1,), (1,)), ((), ())),
        preferred_element_type=jnp.float32)
    emb = emb_ref[...]
    a_i = (jnp.sum(xl * atti_ref[...], axis=1)
           + jnp.sum(emb * attemi_ref[...], axis=1))
    a_j = (jnp.sum(xl * attj_ref[...], axis=1)
           + jnp.sum(emb * attemj_ref[...], axis=1))
    a_i_blk = jax.lax.dynamic_slice(a_i, (r * _BLKR,), (_BLKR,))
    s = a_i_blk[:, None] + a_j[None, :]
    s = jnp.where(s >= 0, s, 0.2 * s)
    s = s + bias_ref[...]
    m = jnp.max(s, axis=1, keepdims=True)
    e = jnp.exp(s - m)
    denom = jnp.sum(e, axis=1, keepdims=True) + 1e-16
    p = e / denom
    agg = jax.lax.dot_general(
        p, xl, (((1,), (0,)), ((), ())),
        preferred_element_type=jnp.float32)
    outv = agg + gb_ref[...]
    out_ref[0] = outv
    rows = r * _BLKR + jax.lax.broadcasted_iota(jnp.int32, (_BLKR, 1), 0)
    valid = rows < _N
    ov = jnp.where(valid, outv, 0.0)
    s1_ref[0, 0] = jnp.sum(ov, axis=0)
    s2_ref[0, 0] = jnp.sum(jnp.where(valid, outv * outv, 0.0), axis=0)


def _bn1_kernel(out_ref, s1_ref, s2_ref, g1_ref, b1_ref, emb_ref,
                xo_ref, t1_ref, t2_ref):
    r = pl.program_id(1)
    mu = jnp.sum(s1_ref[...], axis=(0, 1)) / _CNT
    var = jnp.sum(s2_ref[...], axis=(0, 1)) / _CNT - mu * mu
    istd = jax.lax.rsqrt(var + _EPS)
    h = (out_ref[0] - mu[None, :]) * istd[None, :] * g1_ref[...] + b1_ref[...]
    h = jnp.maximum(h, 0.0)
    xo = h * emb_ref[...]
    xo_ref[0] = xo
    rows = r * _BLKR + jax.lax.broadcasted_iota(jnp.int32, (_BLKR, 1), 0)
    valid = rows < _N
    xv = jnp.where(valid, xo, 0.0)
    t1_ref[0, 0] = jnp.sum(xv, axis=0)
    t2_ref[0, 0] = jnp.sum(jnp.where(valid, xo * xo, 0.0), axis=0)


def _bn2_kernel(xo_ref, t1_ref, t2_ref, g2_ref, b2_ref, ow_ref, ob_ref,
                pred_ref):
    mu = jnp.sum(t1_ref[...], axis=(0, 1)) / _CNT
    var = jnp.sum(t2_ref[...], axis=(0, 1)) / _CNT - mu * mu
    istd = jax.lax.rsqrt(var + _EPS)
    y = (xo_ref[0] - mu[None, :]) * istd[None, :] * g2_ref[...] + b2_ref[...]
    y = jnp.maximum(y, 0.0)
    p = jnp.sum(y * ow_ref[...], axis=1) + ob_ref[0, 0]
    pred_ref[0, 0, :] = p


def kernel(data, org_edge_index, embedding_weight, lin_W, att_i, att_j,
           att_em_i, att_em_j, gnn_bias, bn1_gamma, bn1_beta, bn2_gamma,
           bn2_beta, out_W, out_b):
    del org_edge_index
    f32 = jnp.float32
    emb_p = jnp.zeros((_NP, _D), f32).at[:_N].set(embedding_weight)
    data_p = jnp.zeros((_B, _NP, _IN), f32).at[:, :_N].set(data)

    bias = pl.pallas_call(
        _mask_kernel,
        grid=(_NP // _BLK,),
        in_specs=[
            pl.BlockSpec((_NP, _D), lambda r: (0, 0)),
            pl.BlockSpec((_BLK, _D), lambda r: (r, 0)),
        ],
        out_specs=pl.BlockSpec((_BLK, _NP), lambda r: (r, 0)),
        out_shape=jax.ShapeDtypeStruct((_NP, _NP), f32),
    )(emb_p, emb_p)

    atti = att_i.reshape(1, _D)
    attj = att_j.reshape(1, _D)
    attemi = att_em_i.reshape(1, _D)
    attemj = att_em_j.reshape(1, _D)
    gb = gnn_bias.reshape(1, _D)

    out, s1, s2 = pl.pallas_call(
        _attn_kernel,
        grid=(_B, _NR),
        in_specs=[
            pl.BlockSpec((1, _NP, _IN), lambda b, r: (b, 0, 0)),
            pl.BlockSpec((_D, _IN), lambda b, r: (0, 0)),
            pl.BlockSpec((_NP, _D), lambda b, r: (0, 0)),
            pl.BlockSpec((1, _D), lambda b, r: (0, 0)),
            pl.BlockSpec((1, _D), lambda b, r: (0, 0)),
            pl.BlockSpec((1, _D), lambda b, r: (0, 0)),
            pl.BlockSpec((1, _D), lambda b, r: (0, 0)),
            pl.BlockSpec((1, _D), lambda b, r: (0, 0)),
            pl.BlockSpec((_BLKR, _NP), lambda b, r: (r, 0)),
        ],
        out_specs=[
            pl.BlockSpec((1, _BLKR, _D), lambda b, r: (b, r, 0)),
            pl.BlockSpec((1, 1, _D), lambda b, r: (b * _NR + r, 0, 0)),
            pl.BlockSpec((1, 1, _D), lambda b, r: (b * _NR + r, 0, 0)),
        ],
        out_shape=[
            jax.ShapeDtypeStruct((_B, _NP, _D), f32),
            jax.ShapeDtypeStruct((_B * _NR, 1, _D), f32),
            jax.ShapeDtypeStruct((_B * _NR, 1, _D), f32),
        ],
    )(data_p, lin_W, emb_p, atti, attj, attemi, attemj, gb, bias)

    xo, t1, t2 = pl.pallas_call(
        _bn1_kernel,
        grid=(_B, _NR),
        in_specs=[
            pl.BlockSpec((1, _BLKR, _D), lambda b, r: (b, r, 0)),
            pl.BlockSpec((_B * _NR, 1, _D), lambda b, r: (0, 0, 0)),
            pl.BlockSpec((_B * _NR, 1, _D), lambda b, r: (0, 0, 0)),
            pl.BlockSpec((1, _D), lambda b, r: (0, 0)),
            pl.BlockSpec((1, _D), lambda b, r: (0, 0)),
            pl.BlockSpec((_BLKR, _D), lambda b, r: (r, 0)),
        ],
        out_specs=[
            pl.BlockSpec((1, _BLKR, _D), lambda b, r: (b, r, 0)),
            pl.BlockSpec((1, 1, _D), lambda b, r: (b * _NR + r, 0, 0)),
            pl.BlockSpec((1, 1, _D), lambda b, r: (b * _NR + r, 0, 0)),
        ],
        out_shape=[
            jax.ShapeDtypeStruct((_B, _NP, _D), f32),
            jax.ShapeDtypeStruct((_B * _NR, 1, _D), f32),
            jax.ShapeDtypeStruct((_B * _NR, 1, _D), f32),
        ],
    )(out, s1, s2, bn1_gamma.reshape(1, _D), bn1_beta.reshape(1, _D), emb_p)

    pred = pl.pallas_call(
        _bn2_kernel,
        grid=(_B,),
        in_specs=[
            pl.BlockSpec((1, _NP, _D), lambda b: (b, 0, 0)),
            pl.BlockSpec((_B * _NR, 1, _D), lambda b: (0, 0, 0)),
            pl.BlockSpec((_B * _NR, 1, _D), lambda b: (0, 0, 0)),
            pl.BlockSpec((1, _D), lambda b: (0, 0)),
            pl.BlockSpec((1, _D), lambda b: (0, 0)),
            pl.BlockSpec((1, _D), lambda b: (0, 0)),
            pl.BlockSpec((1, 1), lambda b: (0, 0)),
        ],
        out_specs=pl.BlockSpec((1, 1, _NP), lambda b: (b, 0, 0)),
        out_shape=jax.ShapeDtypeStruct((_B, 1, _NP), f32),
    )(xo, t1, t2, bn2_gamma.reshape(1, _D), bn2_beta.reshape(1, _D),
      out_W.reshape(1, _D), out_b.reshape(1, 1))

    return pred.reshape(_B, _NP)[:, :_N]


# speedup vs baseline: 50.1715x; 50.1715x over previous
import jax
import jax.numpy as jnp
from jax.experimental import pallas as pl

_N = 1000
_NP = 1024
_D = 64
_IN = 16
_K = 20
_B = 32
_BLK = 128
_BLKR = 256
_NR = _NP // _BLKR
_CNT = float(_B * _N)
_EPS = 1e-5


def _mask_kernel(wf_ref, wb_ref, bias_ref):
    r = pl.program_id(0)
    w = wf_ref[...]
    wb = wb_ref[...]
    nrm2 = jnp.maximum(jnp.sum(w * w, axis=1), 1e-12)
    n_full = jnp.sqrt(nrm2)
    nb2 = jnp.maximum(jnp.sum(wb * wb, axis=1), 1e-12)
    n_blk = jnp.sqrt(nb2)
    cos = jax.lax.dot_general(
        wb, w, (((1,), (1,)), ((), ())),
        preferred_element_type=jnp.float32)
    cos = cos / (n_blk[:, None] * n_full[None, :])
    col = jax.lax.broadcasted_iota(jnp.int32, (_BLK, _NP), 1)
    rowg = r * _BLK + jax.lax.broadcasted_iota(jnp.int32, (_BLK, _NP), 0)
    cur = jnp.where(col < _N, cos, -1e9)
    selected = col == rowg
    for _ in range(_K):
        m = jnp.max(cur, axis=1, keepdims=True)
        eq = cur == m
        idx = jnp.min(jnp.where(eq, col, _NP), axis=1, keepdims=True)
        sel = col == idx
        selected = jnp.logical_or(selected, sel)
        cur = jnp.where(sel, -2e9, cur)
    bias_ref[...] = jnp.where(selected, 0.0, -1e30).astype(jnp.float32)


def _attn_kernel(data_ref, datab_ref, lw_ref, emb_ref, embb_ref, atti_ref,
                 attj_ref, attemi_ref, attemj_ref, gb_ref, bias_ref,
                 out_ref, s1_ref, s2_ref):
    r = pl.program_id(1)
    db = data_ref[0]
    xl = jax.lax.dot_general(
        db, lw_ref[...], (((1,), (1,)), ((), ())),
        preferred_element_type=jnp.float32)
    emb = emb_ref[...]
    a_j = (jnp.sum(xl * attj_ref[...], axis=1)
           + jnp.sum(emb * attemj_ref[...], axis=1))
    xlb = jax.lax.dot_general(
        datab_ref[0], lw_ref[...], (((1,), (1,)), ((), ())),
        preferred_element_type=jnp.float32)
    a_i_blk = (jnp.sum(xlb * atti_ref[...], axis=1)
               + jnp.sum(embb_ref[...] * attemi_ref[...], axis=1))
    s = a_i_blk[:, None] + a_j[None, :]
    s = jnp.where(s >= 0, s, 0.2 * s)
    s = s + bias_ref[...]
    m = jnp.max(s, axis=1, keepdims=True)
    e = jnp.exp(s - m)
    denom = jnp.sum(e, axis=1, keepdims=True) + 1e-16
    p = e / denom
    agg = jax.lax.dot_general(
        p, xl, (((1,), (0,)), ((), ())),
        preferred_element_type=jnp.float32)
    outv = agg + gb_ref[...]
    out_ref[0] = outv
    rows = r * _BLKR + jax.lax.broadcasted_iota(jnp.int32, (_BLKR, 1), 0)
    valid = rows < _N
    ov = jnp.where(valid, outv, 0.0)
    s1_ref[0, 0] = jnp.sum(ov, axis=0)
    s2_ref[0, 0] = jnp.sum(jnp.where(valid, outv * outv, 0.0), axis=0)


def _bn1_kernel(out_ref, s1_ref, s2_ref, g1_ref, b1_ref, emb_ref,
                xo_ref, t1_ref, t2_ref):
    r = pl.program_id(1)
    mu = jnp.sum(s1_ref[...], axis=(0, 1)) / _CNT
    var = jnp.sum(s2_ref[...], axis=(0, 1)) / _CNT - mu * mu
    istd = jax.lax.rsqrt(var + _EPS)
    h = (out_ref[0] - mu[None, :]) * istd[None, :] * g1_ref[...] + b1_ref[...]
    h = jnp.maximum(h, 0.0)
    xo = h * emb_ref[...]
    xo_ref[0] = xo
    rows = r * _BLKR + jax.lax.broadcasted_iota(jnp.int32, (_BLKR, 1), 0)
    valid = rows < _N
    xv = jnp.where(valid, xo, 0.0)
    t1_ref[0, 0] = jnp.sum(xv, axis=0)
    t2_ref[0, 0] = jnp.sum(jnp.where(valid, xo * xo, 0.0), axis=0)


def _bn2_kernel(xo_ref, t1_ref, t2_ref, g2_ref, b2_ref, ow_ref, ob_ref,
                pred_ref):
    mu = jnp.sum(t1_ref[...], axis=(0, 1)) / _CNT
    var = jnp.sum(t2_ref[...], axis=(0, 1)) / _CNT - mu * mu
    istd = jax.lax.rsqrt(var + _EPS)
    y = (xo_ref[0] - mu[None, :]) * istd[None, :] * g2_ref[...] + b2_ref[...]
    y = jnp.maximum(y, 0.0)
    p = jnp.sum(y * ow_ref[...], axis=1) + ob_ref[0, 0]
    pred_ref[0, 0, :] = p


def kernel(data, org_edge_index, embedding_weight, lin_W, att_i, att_j,
           att_em_i, att_em_j, gnn_bias, bn1_gamma, bn1_beta, bn2_gamma,
           bn2_beta, out_W, out_b):
    del org_edge_index
    f32 = jnp.float32
    emb_p = jnp.zeros((_NP, _D), f32).at[:_N].set(embedding_weight)
    data_p = jnp.zeros((_B, _NP, _IN), f32).at[:, :_N].set(data)

    bias = pl.pallas_call(
        _mask_kernel,
        grid=(_NP // _BLK,),
        in_specs=[
            pl.BlockSpec((_NP, _D), lambda r: (0, 0)),
            pl.BlockSpec((_BLK, _D), lambda r: (r, 0)),
        ],
        out_specs=pl.BlockSpec((_BLK, _NP), lambda r: (r, 0)),
        out_shape=jax.ShapeDtypeStruct((_NP, _NP), f32),
    )(emb_p, emb_p)

    atti = att_i.reshape(1, _D)
    attj = att_j.reshape(1, _D)
    attemi = att_em_i.reshape(1, _D)
    attemj = att_em_j.reshape(1, _D)
    gb = gnn_bias.reshape(1, _D)

    out, s1, s2 = pl.pallas_call(
        _attn_kernel,
        grid=(_B, _NR),
        in_specs=[
            pl.BlockSpec((1, _NP, _IN), lambda b, r: (b, 0, 0)),
            pl.BlockSpec((1, _BLKR, _IN), lambda b, r: (b, r, 0)),
            pl.BlockSpec((_D, _IN), lambda b, r: (0, 0)),
            pl.BlockSpec((_NP, _D), lambda b, r: (0, 0)),
            pl.BlockSpec((_BLKR, _D), lambda b, r: (r, 0)),
            pl.BlockSpec((1, _D), lambda b, r: (0, 0)),
            pl.BlockSpec((1, _D), lambda b, r: (0, 0)),
            pl.BlockSpec((1, _D), lambda b, r: (0, 0)),
            pl.BlockSpec((1, _D), lambda b, r: (0, 0)),
            pl.BlockSpec((1, _D), lambda b, r: (0, 0)),
            pl.BlockSpec((_BLKR, _NP), lambda b, r: (r, 0)),
        ],
        out_specs=[
            pl.BlockSpec((1, _BLKR, _D), lambda b, r: (b, r, 0)),
            pl.BlockSpec((1, 1, _D), lambda b, r: (b * _NR + r, 0, 0)),
            pl.BlockSpec((1, 1, _D), lambda b, r: (b * _NR + r, 0, 0)),
        ],
        out_shape=[
            jax.ShapeDtypeStruct((_B, _NP, _D), f32),
            jax.ShapeDtypeStruct((_B * _NR, 1, _D), f32),
            jax.ShapeDtypeStruct((_B * _NR, 1, _D), f32),
        ],
    )(data_p, data_p, lin_W, emb_p, emb_p, atti, attj, attemi, attemj, gb,
      bias)

    xo, t1, t2 = pl.pallas_call(
        _bn1_kernel,
        grid=(_B, _NR),
        in_specs=[
            pl.BlockSpec((1, _BLKR, _D), lambda b, r: (b, r, 0)),
            pl.BlockSpec((_B * _NR, 1, _D), lambda b, r: (0, 0, 0)),
            pl.BlockSpec((_B * _NR, 1, _D), lambda b, r: (0, 0, 0)),
            pl.BlockSpec((1, _D), lambda b, r: (0, 0)),
            pl.BlockSpec((1, _D), lambda b, r: (0, 0)),
            pl.BlockSpec((_BLKR, _D), lambda b, r: (r, 0)),
        ],
        out_specs=[
            pl.BlockSpec((1, _BLKR, _D), lambda b, r: (b, r, 0)),
            pl.BlockSpec((1, 1, _D), lambda b, r: (b * _NR + r, 0, 0)),
            pl.BlockSpec((1, 1, _D), lambda b, r: (b * _NR + r, 0, 0)),
        ],
        out_shape=[
            jax.ShapeDtypeStruct((_B, _NP, _D), f32),
            jax.ShapeDtypeStruct((_B * _NR, 1, _D), f32),
            jax.ShapeDtypeStruct((_B * _NR, 1, _D), f32),
        ],
    )(out, s1, s2, bn1_gamma.reshape(1, _D), bn1_beta.reshape(1, _D), emb_p)

    pred = pl.pallas_call(
        _bn2_kernel,
        grid=(_B,),
        in_specs=[
            pl.BlockSpec((1, _NP, _D), lambda b: (b, 0, 0)),
            pl.BlockSpec((_B * _NR, 1, _D), lambda b: (0, 0, 0)),
            pl.BlockSpec((_B * _NR, 1, _D), lambda b: (0, 0, 0)),
            pl.BlockSpec((1, _D), lambda b: (0, 0)),
            pl.BlockSpec((1, _D), lambda b: (0, 0)),
            pl.BlockSpec((1, _D), lambda b: (0, 0)),
            pl.BlockSpec((1, 1), lambda b: (0, 0)),
        ],
        out_specs=pl.BlockSpec((1, 1, _NP), lambda b: (b, 0, 0)),
        out_shape=jax.ShapeDtypeStruct((_B, 1, _NP), f32),
    )(xo, t1, t2, bn2_gamma.reshape(1, _D), bn2_beta.reshape(1, _D),
      out_W.reshape(1, _D), out_b.reshape(1, 1))

    return pred.reshape(_B, _NP)[:, :_N]


# speedup vs baseline: 93.3442x; 1.8605x over previous
import jax
import jax.numpy as jnp
from jax.experimental import pallas as pl

_N = 1000
_NP = 1024
_D = 64
_IN = 16
_K = 20
_B = 32
_BLK = 128
_CNT = float(_B * _N)
_EPS = 1e-5


def _mask_kernel(wf_ref, wb_ref, bias_ref):
    r = pl.program_id(0)
    w = wf_ref[...]
    wb = wb_ref[...]
    nrm2 = jnp.maximum(jnp.sum(w * w, axis=1), 1e-12)
    n_full = jnp.sqrt(nrm2)
    nb2 = jnp.maximum(jnp.sum(wb * wb, axis=1), 1e-12)
    n_blk = jnp.sqrt(nb2)
    cos = jax.lax.dot_general(
        wb, w, (((1,), (1,)), ((), ())),
        preferred_element_type=jnp.float32)
    cos = cos / (n_blk[:, None] * n_full[None, :])
    col = jax.lax.broadcasted_iota(jnp.int32, (_BLK, _NP), 1)
    rowg = r * _BLK + jax.lax.broadcasted_iota(jnp.int32, (_BLK, _NP), 0)
    cur = jnp.where(col < _N, cos, -1e9)
    selected = col == rowg
    for _ in range(_K):
        m = jnp.max(cur, axis=1, keepdims=True)
        eq = cur == m
        idx = jnp.min(jnp.where(eq, col, _NP), axis=1, keepdims=True)
        sel = col == idx
        selected = jnp.logical_or(selected, sel)
        cur = jnp.where(sel, -2e9, cur)
    bias_ref[...] = jnp.where(selected, 0.0, -1e30).astype(jnp.float32)


def _attn_kernel(data_ref, lw_ref, emb_ref, atti_ref, attj_ref, attemi_ref,
                 attemj_ref, gb_ref, bias_ref, out_ref, s1_ref, s2_ref):
    db = data_ref[0]
    xl = jax.lax.dot_general(
        db, lw_ref[...], (((1,), (1,)), ((), ())),
        preferred_element_type=jnp.float32)
    emb = emb_ref[...]
    aj_row = (jax.lax.dot_general(
                  attj_ref[...], xl, (((1,), (1,)), ((), ())),
                  preferred_element_type=jnp.float32)
              + jax.lax.dot_general(
                  attemj_ref[...], emb, (((1,), (1,)), ((), ())),
                  preferred_element_type=jnp.float32))
    a_i = (jnp.sum(xl * atti_ref[...], axis=1, keepdims=True)
           + jnp.sum(emb * attemi_ref[...], axis=1, keepdims=True))
    s = a_i + aj_row
    s = jnp.maximum(s, 0.2 * s)
    e = jnp.exp(s + bias_ref[...])
    denom = jnp.sum(e, axis=1, keepdims=True) + 1e-16
    u = jax.lax.dot_general(
        e, xl, (((1,), (0,)), ((), ())),
        preferred_element_type=jnp.float32)
    outv = u / denom + gb_ref[...]
    out_ref[0] = outv
    rows = jax.lax.broadcasted_iota(jnp.int32, (_NP, 1), 0)
    valid = rows < _N
    ov = jnp.where(valid, outv, 0.0)
    s1_ref[0, 0] = jnp.sum(ov, axis=0)
    s2_ref[0, 0] = jnp.sum(jnp.where(valid, outv * outv, 0.0), axis=0)


def _stats_kernel(s1_ref, s2_ref, g_ref, b_ref, scale_ref, shift_ref):
    mu = jnp.sum(s1_ref[...], axis=(0, 1)) / _CNT
    var = jnp.sum(s2_ref[...], axis=(0, 1)) / _CNT - mu * mu
    istd = jax.lax.rsqrt(var + _EPS)
    scale = istd * g_ref[0]
    scale_ref[0, :] = scale
    shift_ref[0, :] = b_ref[0] - mu * scale


def _bn1_kernel(out_ref, sc_ref, sh_ref, emb_ref, xo_ref, t1_ref, t2_ref):
    h = out_ref[0] * sc_ref[...] + sh_ref[...]
    h = jnp.maximum(h, 0.0)
    xo = h * emb_ref[...]
    xo_ref[0] = xo
    rows = jax.lax.broadcasted_iota(jnp.int32, (_NP, 1), 0)
    valid = rows < _N
    xv = jnp.where(valid, xo, 0.0)
    t1_ref[0, 0] = jnp.sum(xv, axis=0)
    t2_ref[0, 0] = jnp.sum(jnp.where(valid, xo * xo, 0.0), axis=0)


def _bn2_kernel(xo_ref, sc_ref, sh_ref, ow_ref, ob_ref, pred_ref):
    y = xo_ref[0] * sc_ref[...] + sh_ref[...]
    y = jnp.maximum(y, 0.0)
    p = jnp.sum(y * ow_ref[...], axis=1) + ob_ref[0, 0]
    pred_ref[0, 0, :] = p


def kernel(data, org_edge_index, embedding_weight, lin_W, att_i, att_j,
           att_em_i, att_em_j, gnn_bias, bn1_gamma, bn1_beta, bn2_gamma,
           bn2_beta, out_W, out_b):
    del org_edge_index
    f32 = jnp.float32
    emb_p = jnp.zeros((_NP, _D), f32).at[:_N].set(embedding_weight)
    data_p = jnp.zeros((_B, _NP, _IN), f32).at[:, :_N].set(data)

    bias = pl.pallas_call(
        _mask_kernel,
        grid=(_NP // _BLK,),
        in_specs=[
            pl.BlockSpec((_NP, _D), lambda r: (0, 0)),
            pl.BlockSpec((_BLK, _D), lambda r: (r, 0)),
        ],
        out_specs=pl.BlockSpec((_BLK, _NP), lambda r: (r, 0)),
        out_shape=jax.ShapeDtypeStruct((_NP, _NP), f32),
    )(emb_p, emb_p)

    atti = att_i.reshape(1, _D)
    attj = att_j.reshape(1, _D)
    attemi = att_em_i.reshape(1, _D)
    attemj = att_em_j.reshape(1, _D)
    gb = gnn_bias.reshape(1, _D)

    vec = lambda: pl.BlockSpec((1, _D), lambda b: (0, 0))
    out, s1, s2 = pl.pallas_call(
        _attn_kernel,
        grid=(_B,),
        in_specs=[
            pl.BlockSpec((1, _NP, _IN), lambda b: (b, 0, 0)),
            pl.BlockSpec((_D, _IN), lambda b: (0, 0)),
            pl.BlockSpec((_NP, _D), lambda b: (0, 0)),
            vec(), vec(), vec(), vec(), vec(),
            pl.BlockSpec((_NP, _NP), lambda b: (0, 0)),
        ],
        out_specs=[
            pl.BlockSpec((1, _NP, _D), lambda b: (b, 0, 0)),
            pl.BlockSpec((1, 1, _D), lambda b: (b, 0, 0)),
            pl.BlockSpec((1, 1, _D), lambda b: (b, 0, 0)),
        ],
        out_shape=[
            jax.ShapeDtypeStruct((_B, _NP, _D), f32),
            jax.ShapeDtypeStruct((_B, 1, _D), f32),
            jax.ShapeDtypeStruct((_B, 1, _D), f32),
        ],
    )(data_p, lin_W, emb_p, atti, attj, attemi, attemj, gb, bias)

    def _stats(s1v, s2v, g, b):
        return pl.pallas_call(
            _stats_kernel,
            grid=(1,),
            in_specs=[
                pl.BlockSpec((_B, 1, _D), lambda i: (0, 0, 0)),
                pl.BlockSpec((_B, 1, _D), lambda i: (0, 0, 0)),
                pl.BlockSpec((1, _D), lambda i: (0, 0)),
                pl.BlockSpec((1, _D), lambda i: (0, 0)),
            ],
            out_specs=[
                pl.BlockSpec((1, _D), lambda i: (0, 0)),
                pl.BlockSpec((1, _D), lambda i: (0, 0)),
            ],
            out_shape=[
                jax.ShapeDtypeStruct((1, _D), f32),
                jax.ShapeDtypeStruct((1, _D), f32),
            ],
        )(s1v, s2v, g.reshape(1, _D), b.reshape(1, _D))

    sc1, sh1 = _stats(s1, s2, bn1_gamma, bn1_beta)

    xo, t1, t2 = pl.pallas_call(
        _bn1_kernel,
        grid=(_B,),
        in_specs=[
            pl.BlockSpec((1, _NP, _D), lambda b: (b, 0, 0)),
            pl.BlockSpec((1, _D), lambda b: (0, 0)),
            pl.BlockSpec((1, _D), lambda b: (0, 0)),
            pl.BlockSpec((_NP, _D), lambda b: (0, 0)),
        ],
        out_specs=[
            pl.BlockSpec((1, _NP, _D), lambda b: (b, 0, 0)),
            pl.BlockSpec((1, 1, _D), lambda b: (b, 0, 0)),
            pl.BlockSpec((1, 1, _D), lambda b: (b, 0, 0)),
        ],
        out_shape=[
            jax.ShapeDtypeStruct((_B, _NP, _D), f32),
            jax.ShapeDtypeStruct((_B, 1, _D), f32),
            jax.ShapeDtypeStruct((_B, 1, _D), f32),
        ],
    )(out, sc1, sh1, emb_p)

    sc2, sh2 = _stats(t1, t2, bn2_gamma, bn2_beta)

    pred = pl.pallas_call(
        _bn2_kernel,
        grid=(_B,),
        in_specs=[
            pl.BlockSpec((1, _NP, _D), lambda b: (b, 0, 0)),
            pl.BlockSpec((1, _D), lambda b: (0, 0)),
            pl.BlockSpec((1, _D), lambda b: (0, 0)),
            pl.BlockSpec((1, _D), lambda b: (0, 0)),
            pl.BlockSpec((1, 1), lambda b: (0, 0)),
        ],
        out_specs=pl.BlockSpec((1, 1, _NP), lambda b: (b, 0, 0)),
        out_shape=jax.ShapeDtypeStruct((_B, 1, _NP), f32),
    )(xo, sc2, sh2, out_W.reshape(1, _D), out_b.reshape(1, 1))

    return pred.reshape(_B, _NP)[:, :_N]


# speedup vs baseline: 119.1622x; 1.2766x over previous
import jax
import jax.numpy as jnp
from jax.experimental import pallas as pl

_N = 1000
_NP = 1024
_D = 64
_IN = 16
_K = 20
_B = 32
_BLK = 128
_CNT = float(_B * _N)
_EPS = 1e-5


def _mask_kernel(wf_ref, wb_ref, bias_ref):
    r = pl.program_id(0)
    w = wf_ref[...]
    wb = wb_ref[...]
    nrm2 = jnp.maximum(jnp.sum(w * w, axis=1), 1e-12)
    n_full = jnp.sqrt(nrm2)
    nb2 = jnp.maximum(jnp.sum(wb * wb, axis=1), 1e-12)
    n_blk = jnp.sqrt(nb2)
    cos = jax.lax.dot_general(
        wb, w, (((1,), (1,)), ((), ())),
        preferred_element_type=jnp.float32)
    cos = cos / (n_blk[:, None] * n_full[None, :])
    col = jax.lax.broadcasted_iota(jnp.int32, (_BLK, _NP), 1)
    rowg = r * _BLK + jax.lax.broadcasted_iota(jnp.int32, (_BLK, _NP), 0)
    cmask = jnp.where(col < _N, cos, -1e9)
    cur = cmask
    v_k = None
    for _ in range(_K):
        v_k = jnp.max(cur, axis=1, keepdims=True)
        cur = jnp.where(cur == v_k, -2e9, cur)
    selected = jnp.logical_or(cmask >= v_k, col == rowg)
    bias_ref[...] = jnp.where(selected, 0.0, -1e30).astype(jnp.float32)


def _attn_kernel(data_ref, lw_ref, emb_ref, atti_ref, attj_ref, attemi_ref,
                 attemj_ref, gb_ref, bias_ref, out_ref, s1_ref, s2_ref):
    db = data_ref[0]
    xl = jax.lax.dot_general(
        db, lw_ref[...], (((1,), (1,)), ((), ())),
        preferred_element_type=jnp.float32)
    emb = emb_ref[...]
    aj_row = (jax.lax.dot_general(
                  attj_ref[...], xl, (((1,), (1,)), ((), ())),
                  preferred_element_type=jnp.float32)
              + jax.lax.dot_general(
                  attemj_ref[...], emb, (((1,), (1,)), ((), ())),
                  preferred_element_type=jnp.float32))
    a_i = (jnp.sum(xl * atti_ref[...], axis=1, keepdims=True)
           + jnp.sum(emb * attemi_ref[...], axis=1, keepdims=True))
    s = a_i + aj_row
    s = jnp.maximum(s, 0.2 * s)
    e = jnp.exp(s + bias_ref[...])
    denom = jnp.sum(e, axis=1, keepdims=True) + 1e-16
    u = jax.lax.dot_general(
        e, xl, (((1,), (0,)), ((), ())),
        preferred_element_type=jnp.float32)
    outv = u / denom + gb_ref[...]
    out_ref[0] = outv
    rows = jax.lax.broadcasted_iota(jnp.int32, (_NP, 1), 0)
    valid = rows < _N
    ov = jnp.where(valid, outv, 0.0)
    s1_ref[0, 0] = jnp.sum(ov, axis=0)
    s2_ref[0, 0] = jnp.sum(jnp.where(valid, outv * outv, 0.0), axis=0)


def _stats_kernel(s1_ref, s2_ref, g_ref, b_ref, scale_ref, shift_ref):
    mu = jnp.sum(s1_ref[...], axis=(0, 1)) / _CNT
    var = jnp.sum(s2_ref[...], axis=(0, 1)) / _CNT - mu * mu
    istd = jax.lax.rsqrt(var + _EPS)
    scale = istd * g_ref[0]
    scale_ref[0, :] = scale
    shift_ref[0, :] = b_ref[0] - mu * scale


def _bn1_kernel(out_ref, sc_ref, sh_ref, emb_ref, t1_ref, t2_ref):
    h = out_ref[0] * sc_ref[...] + sh_ref[...]
    h = jnp.maximum(h, 0.0)
    xo = h * emb_ref[...]
    rows = jax.lax.broadcasted_iota(jnp.int32, (_NP, 1), 0)
    valid = rows < _N
    xv = jnp.where(valid, xo, 0.0)
    t1_ref[0, 0] = jnp.sum(xv, axis=0)
    t2_ref[0, 0] = jnp.sum(jnp.where(valid, xo * xo, 0.0), axis=0)


def _bn2_kernel(out_ref, sc1_ref, sh1_ref, emb_ref, sc_ref, sh_ref, ow_ref,
                ob_ref, pred_ref):
    h = out_ref[0] * sc1_ref[...] + sh1_ref[...]
    h = jnp.maximum(h, 0.0)
    xo = h * emb_ref[...]
    y = xo * sc_ref[...] + sh_ref[...]
    y = jnp.maximum(y, 0.0)
    p = jax.lax.dot_general(
        ow_ref[...], y, (((1,), (1,)), ((), ())),
        preferred_element_type=jnp.float32)
    pred_ref[0] = p + ob_ref[0, 0]


def kernel(data, org_edge_index, embedding_weight, lin_W, att_i, att_j,
           att_em_i, att_em_j, gnn_bias, bn1_gamma, bn1_beta, bn2_gamma,
           bn2_beta, out_W, out_b):
    del org_edge_index
    f32 = jnp.float32
    emb_p = jnp.zeros((_NP, _D), f32).at[:_N].set(embedding_weight)
    data_p = jnp.zeros((_B, _NP, _IN), f32).at[:, :_N].set(data)

    bias = pl.pallas_call(
        _mask_kernel,
        grid=(_NP // _BLK,),
        in_specs=[
            pl.BlockSpec((_NP, _D), lambda r: (0, 0)),
            pl.BlockSpec((_BLK, _D), lambda r: (r, 0)),
        ],
        out_specs=pl.BlockSpec((_BLK, _NP), lambda r: (r, 0)),
        out_shape=jax.ShapeDtypeStruct((_NP, _NP), f32),
    )(emb_p, emb_p)

    atti = att_i.reshape(1, _D)
    attj = att_j.reshape(1, _D)
    attemi = att_em_i.reshape(1, _D)
    attemj = att_em_j.reshape(1, _D)
    gb = gnn_bias.reshape(1, _D)

    vec = lambda: pl.BlockSpec((1, _D), lambda b: (0, 0))
    out, s1, s2 = pl.pallas_call(
        _attn_kernel,
        grid=(_B,),
        in_specs=[
            pl.BlockSpec((1, _NP, _IN), lambda b: (b, 0, 0)),
            pl.BlockSpec((_D, _IN), lambda b: (0, 0)),
            pl.BlockSpec((_NP, _D), lambda b: (0, 0)),
            vec(), vec(), vec(), vec(), vec(),
            pl.BlockSpec((_NP, _NP), lambda b: (0, 0)),
        ],
        out_specs=[
            pl.BlockSpec((1, _NP, _D), lambda b: (b, 0, 0)),
            pl.BlockSpec((1, 1, _D), lambda b: (b, 0, 0)),
            pl.BlockSpec((1, 1, _D), lambda b: (b, 0, 0)),
        ],
        out_shape=[
            jax.ShapeDtypeStruct((_B, _NP, _D), f32),
            jax.ShapeDtypeStruct((_B, 1, _D), f32),
            jax.ShapeDtypeStruct((_B, 1, _D), f32),
        ],
    )(data_p, lin_W, emb_p, atti, attj, attemi, attemj, gb, bias)

    def _stats(s1v, s2v, g, b):
        return pl.pallas_call(
            _stats_kernel,
            grid=(1,),
            in_specs=[
                pl.BlockSpec((_B, 1, _D), lambda i: (0, 0, 0)),
                pl.BlockSpec((_B, 1, _D), lambda i: (0, 0, 0)),
                pl.BlockSpec((1, _D), lambda i: (0, 0)),
                pl.BlockSpec((1, _D), lambda i: (0, 0)),
            ],
            out_specs=[
                pl.BlockSpec((1, _D), lambda i: (0, 0)),
                pl.BlockSpec((1, _D), lambda i: (0, 0)),
            ],
            out_shape=[
                jax.ShapeDtypeStruct((1, _D), f32),
                jax.ShapeDtypeStruct((1, _D), f32),
            ],
        )(s1v, s2v, g.reshape(1, _D), b.reshape(1, _D))

    sc1, sh1 = _stats(s1, s2, bn1_gamma, bn1_beta)

    t1, t2 = pl.pallas_call(
        _bn1_kernel,
        grid=(_B,),
        in_specs=[
            pl.BlockSpec((1, _NP, _D), lambda b: (b, 0, 0)),
            pl.BlockSpec((1, _D), lambda b: (0, 0)),
            pl.BlockSpec((1, _D), lambda b: (0, 0)),
            pl.BlockSpec((_NP, _D), lambda b: (0, 0)),
        ],
        out_specs=[
            pl.BlockSpec((1, 1, _D), lambda b: (b, 0, 0)),
            pl.BlockSpec((1, 1, _D), lambda b: (b, 0, 0)),
        ],
        out_shape=[
            jax.ShapeDtypeStruct((_B, 1, _D), f32),
            jax.ShapeDtypeStruct((_B, 1, _D), f32),
        ],
    )(out, sc1, sh1, emb_p)

    sc2, sh2 = _stats(t1, t2, bn2_gamma, bn2_beta)

    pred = pl.pallas_call(
        _bn2_kernel,
        grid=(_B,),
        in_specs=[
            pl.BlockSpec((1, _NP, _D), lambda b: (b, 0, 0)),
            pl.BlockSpec((1, _D), lambda b: (0, 0)),
            pl.BlockSpec((1, _D), lambda b: (0, 0)),
            pl.BlockSpec((_NP, _D), lambda b: (0, 0)),
            pl.BlockSpec((1, _D), lambda b: (0, 0)),
            pl.BlockSpec((1, _D), lambda b: (0, 0)),
            pl.BlockSpec((1, _D), lambda b: (0, 0)),
            pl.BlockSpec((1, 1), lambda b: (0, 0)),
        ],
        out_specs=pl.BlockSpec((1, 1, _NP), lambda b: (b, 0, 0)),
        out_shape=jax.ShapeDtypeStruct((_B, 1, _NP), f32),
    )(out, sc1, sh1, emb_p, sc2, sh2, out_W.reshape(1, _D),
      out_b.reshape(1, 1))

    return pred.reshape(_B, _NP)[:, :_N]


# speedup vs baseline: 122.1539x; 1.0251x over previous
import jax
import jax.numpy as jnp
from jax.experimental import pallas as pl
from jax.experimental.pallas import tpu as pltpu

_N = 1000
_NP = 1024
_D = 64
_IN = 16
_K = 20
_B = 32
_BLK = 128
_NB = _NP // _BLK
_CNT = float(_B * _N)
_EPS = 1e-5


def _attn_kernel(emb_ref, embb_ref, data_ref, lw_ref, atti_ref, attj_ref,
                 attemi_ref, attemj_ref, gb_ref, out_ref, s1_ref, s2_ref,
                 bias_scr):
    i = pl.program_id(0)

    @pl.when(i < _NB)
    def _mask_phase():
        w = emb_ref[...]
        wb = embb_ref[...]
        n_full = jnp.sqrt(jnp.maximum(jnp.sum(w * w, axis=1), 1e-12))
        n_blk = jnp.sqrt(jnp.maximum(jnp.sum(wb * wb, axis=1), 1e-12))
        cos = jax.lax.dot_general(
            wb, w, (((1,), (1,)), ((), ())),
            preferred_element_type=jnp.float32)
        cos = cos / (n_blk[:, None] * n_full[None, :])
        col = jax.lax.broadcasted_iota(jnp.int32, (_BLK, _NP), 1)
        rowg = i * _BLK + jax.lax.broadcasted_iota(jnp.int32, (_BLK, _NP), 0)
        cmask = jnp.where(col < _N, cos, -1e9)
        cur = cmask
        v_k = None
        for _ in range(_K):
            v_k = jnp.max(cur, axis=1, keepdims=True)
            cur = jnp.where(cur == v_k, -2e9, cur)
        selected = jnp.logical_or(cmask >= v_k, col == rowg)
        bias_scr[pl.ds(i * _BLK, _BLK), :] = jnp.where(
            selected, 0.0, -1e30).astype(jnp.float32)

    @pl.when(i >= _NB)
    def _attn_phase():
        db = data_ref[0]
        xl = jax.lax.dot_general(
            db, lw_ref[...], (((1,), (1,)), ((), ())),
            preferred_element_type=jnp.float32)
        emb = emb_ref[...]
        aj_row = (jax.lax.dot_general(
                      attj_ref[...], xl, (((1,), (1,)), ((), ())),
                      preferred_element_type=jnp.float32)
                  + jax.lax.dot_general(
                      attemj_ref[...], emb, (((1,), (1,)), ((), ())),
                      preferred_element_type=jnp.float32))
        a_i = (jnp.sum(xl * atti_ref[...], axis=1, keepdims=True)
               + jnp.sum(emb * attemi_ref[...], axis=1, keepdims=True))
        s = a_i + aj_row
        s = jnp.maximum(s, 0.2 * s)
        e = jnp.exp(s + bias_scr[...])
        denom = jnp.sum(e, axis=1, keepdims=True) + 1e-16
        u = jax.lax.dot_general(
            e, xl, (((1,), (0,)), ((), ())),
            preferred_element_type=jnp.float32)
        outv = u / denom + gb_ref[...]
        out_ref[0] = outv
        rows = jax.lax.broadcasted_iota(jnp.int32, (_NP, 1), 0)
        valid = rows < _N
        ov = jnp.where(valid, outv, 0.0)
        s1_ref[0, 0] = jnp.sum(ov, axis=0)
        s2_ref[0, 0] = jnp.sum(jnp.where(valid, outv * outv, 0.0), axis=0)


def _bn1_kernel(s1_ref, s2_ref, g1_ref, b1_ref, out_ref, emb_ref,
                sc_ref, sh_ref, t1_ref, t2_ref, st_scr):
    i = pl.program_id(0)

    @pl.when(i == 0)
    def _stats_phase():
        mu = jnp.sum(s1_ref[...], axis=(0, 1)) / _CNT
        var = jnp.sum(s2_ref[...], axis=(0, 1)) / _CNT - mu * mu
        scale = jax.lax.rsqrt(var + _EPS) * g1_ref[0]
        shift = b1_ref[0] - mu * scale
        st_scr[0, :] = scale
        st_scr[1, :] = shift
        sc_ref[0, :] = scale
        sh_ref[0, :] = shift

    @pl.when(i > 0)
    def _t_phase():
        h = out_ref[0] * st_scr[0, :][None, :] + st_scr[1, :][None, :]
        h = jnp.maximum(h, 0.0)
        xo = h * emb_ref[...]
        rows = jax.lax.broadcasted_iota(jnp.int32, (_NP, 1), 0)
        valid = rows < _N
        xv = jnp.where(valid, xo, 0.0)
        t1_ref[0, 0] = jnp.sum(xv, axis=0)
        t2_ref[0, 0] = jnp.sum(jnp.where(valid, xo * xo, 0.0), axis=0)


def _bn2_kernel(t1_ref, t2_ref, g2_ref, b2_ref, sc1_ref, sh1_ref, out_ref,
                emb_ref, ow_ref, ob_ref, pred_ref, st_scr):
    i = pl.program_id(0)

    @pl.when(i == 0)
    def _stats_phase():
        mu = jnp.sum(t1_ref[...], axis=(0, 1)) / _CNT
        var = jnp.sum(t2_ref[...], axis=(0, 1)) / _CNT - mu * mu
        scale = jax.lax.rsqrt(var + _EPS) * g2_ref[0]
        st_scr[0, :] = scale
        st_scr[1, :] = b2_ref[0] - mu * scale

    @pl.when(i > 0)
    def _apply_phase():
        h = out_ref[0] * sc1_ref[...] + sh1_ref[...]
        h = jnp.maximum(h, 0.0)
        xo = h * emb_ref[...]
        y = xo * st_scr[0, :][None, :] + st_scr[1, :][None, :]
        y = jnp.maximum(y, 0.0)
        p = jax.lax.dot_general(
            ow_ref[...], y, (((1,), (1,)), ((), ())),
            preferred_element_type=jnp.float32)
        pred_ref[0] = p + ob_ref[0, 0]


def kernel(data, org_edge_index, embedding_weight, lin_W, att_i, att_j,
           att_em_i, att_em_j, gnn_bias, bn1_gamma, bn1_beta, bn2_gamma,
           bn2_beta, out_W, out_b):
    del org_edge_index
    f32 = jnp.float32
    emb_p = jnp.zeros((_NP, _D), f32).at[:_N].set(embedding_weight)
    data_p = jnp.zeros((_B, _NP, _IN), f32).at[:, :_N].set(data)

    vec = lambda: pl.BlockSpec((1, _D), lambda i: (0, 0))
    out, s1, s2 = pl.pallas_call(
        _attn_kernel,
        grid=(_NB + _B,),
        in_specs=[
            pl.BlockSpec((_NP, _D), lambda i: (0, 0)),
            pl.BlockSpec((_BLK, _D), lambda i: (jnp.minimum(i, _NB - 1), 0)),
            pl.BlockSpec((1, _NP, _IN),
                         lambda i: (jnp.maximum(i - _NB, 0), 0, 0)),
            pl.BlockSpec((_D, _IN), lambda i: (0, 0)),
            vec(), vec(), vec(), vec(), vec(),
        ],
        out_specs=[
            pl.BlockSpec((1, _NP, _D), lambda i: (jnp.maximum(i - _NB, 0),
                                                  0, 0)),
            pl.BlockSpec((1, 1, _D), lambda i: (jnp.maximum(i - _NB, 0),
                                                0, 0)),
            pl.BlockSpec((1, 1, _D), lambda i: (jnp.maximum(i - _NB, 0),
                                                0, 0)),
        ],
        out_shape=[
            jax.ShapeDtypeStruct((_B, _NP, _D), f32),
            jax.ShapeDtypeStruct((_B, 1, _D), f32),
            jax.ShapeDtypeStruct((_B, 1, _D), f32),
        ],
        scratch_shapes=[pltpu.VMEM((_NP, _NP), f32)],
    )(emb_p, emb_p, data_p, lin_W, att_i.reshape(1, _D), att_j.reshape(1, _D),
      att_em_i.reshape(1, _D), att_em_j.reshape(1, _D),
      gnn_bias.reshape(1, _D))

    sc1, sh1, t1, t2 = pl.pallas_call(
        _bn1_kernel,
        grid=(1 + _B,),
        in_specs=[
            pl.BlockSpec((_B, 1, _D), lambda i: (0, 0, 0)),
            pl.BlockSpec((_B, 1, _D), lambda i: (0, 0, 0)),
            pl.BlockSpec((1, _D), lambda i: (0, 0)),
            pl.BlockSpec((1, _D), lambda i: (0, 0)),
            pl.BlockSpec((1, _NP, _D), lambda i: (jnp.maximum(i - 1, 0),
                                                  0, 0)),
            pl.BlockSpec((_NP, _D), lambda i: (0, 0)),
        ],
        out_specs=[
            pl.BlockSpec((1, _D), lambda i: (0, 0)),
            pl.BlockSpec((1, _D), lambda i: (0, 0)),
            pl.BlockSpec((1, 1, _D), lambda i: (jnp.maximum(i - 1, 0), 0, 0)),
            pl.BlockSpec((1, 1, _D), lambda i: (jnp.maximum(i - 1, 0), 0, 0)),
        ],
        out_shape=[
            jax.ShapeDtypeStruct((1, _D), f32),
            jax.ShapeDtypeStruct((1, _D), f32),
            jax.ShapeDtypeStruct((_B, 1, _D), f32),
            jax.ShapeDtypeStruct((_B, 1, _D), f32),
        ],
        scratch_shapes=[pltpu.VMEM((2, _D), f32)],
    )(s1, s2, bn1_gamma.reshape(1, _D), bn1_beta.reshape(1, _D), out, emb_p)

    pred = pl.pallas_call(
        _bn2_kernel,
        grid=(1 + _B,),
        in_specs=[
            pl.BlockSpec((_B, 1, _D), lambda i: (0, 0, 0)),
            pl.BlockSpec((_B, 1, _D), lambda i: (0, 0, 0)),
            pl.BlockSpec((1, _D), lambda i: (0, 0)),
            pl.BlockSpec((1, _D), lambda i: (0, 0)),
            pl.BlockSpec((1, _D), lambda i: (0, 0)),
            pl.BlockSpec((1, _D), lambda i: (0, 0)),
            pl.BlockSpec((1, _NP, _D), lambda i: (jnp.maximum(i - 1, 0),
                                                  0, 0)),
            pl.BlockSpec((_NP, _D), lambda i: (0, 0)),
            pl.BlockSpec((1, _D), lambda i: (0, 0)),
            pl.BlockSpec((1, 1), lambda i: (0, 0)),
        ],
        out_specs=pl.BlockSpec((1, 1, _NP), lambda i: (jnp.maximum(i - 1, 0),
                                                       0, 0)),
        out_shape=jax.ShapeDtypeStruct((_B, 1, _NP), f32),
        scratch_shapes=[pltpu.VMEM((2, _D), f32)],
    )(t1, t2, bn2_gamma.reshape(1, _D), bn2_beta.reshape(1, _D), sc1, sh1,
      out, emb_p, out_W.reshape(1, _D), out_b.reshape(1, 1))

    return pred.reshape(_B, _NP)[:, :_N]


# speedup vs baseline: 151.4655x; 1.2400x over previous
import jax
import jax.numpy as jnp
from jax.experimental import pallas as pl
from jax.experimental.pallas import tpu as pltpu

_N = 1000
_NP = 1024
_D = 64
_IN = 16
_K = 20
_B = 32
_BLK = 128
_NB = _NP // _BLK
_AB = 2
_GB = 4
_CNT = float(_B * _N)
_EPS = 1e-5


def _attn_kernel(emb_ref, embb_ref, data_ref, lw_ref, atti_ref, attj_ref,
                 attemi_ref, attemj_ref, gb_ref, out_ref, s1_ref, s2_ref,
                 bias_scr):
    i = pl.program_id(0)

    @pl.when(i < _NB)
    def _mask_phase():
        w = emb_ref[...]
        wb = embb_ref[...]
        n_full = jnp.sqrt(jnp.maximum(jnp.sum(w * w, axis=1), 1e-12))
        n_blk = jnp.sqrt(jnp.maximum(jnp.sum(wb * wb, axis=1), 1e-12))
        cos = jax.lax.dot_general(
            wb, w, (((1,), (1,)), ((), ())),
            preferred_element_type=jnp.float32)
        cos = cos / (n_blk[:, None] * n_full[None, :])
        col = jax.lax.broadcasted_iota(jnp.int32, (_BLK, _NP), 1)
        rowg = i * _BLK + jax.lax.broadcasted_iota(jnp.int32, (_BLK, _NP), 0)
        cmask = jnp.where(col < _N, cos, -1e9)
        cur = cmask
        v_k = None
        for _ in range(_K):
            v_k = jnp.max(cur, axis=1, keepdims=True)
            cur = jnp.where(cur == v_k, -2e9, cur)
        selected = jnp.logical_or(cmask >= v_k, col == rowg)
        bias_scr[pl.ds(i * _BLK, _BLK), :] = jnp.where(
            selected, 0.0, -1e30).astype(jnp.float32)

    @pl.when(i >= _NB)
    def _attn_phase():
        emb = emb_ref[...]
        rows = jax.lax.broadcasted_iota(jnp.int32, (_NP, 1), 0)
        valid = rows < _N
        for k in range(_AB):
            db = data_ref[k]
            xl = jax.lax.dot_general(
                db, lw_ref[...], (((1,), (1,)), ((), ())),
                preferred_element_type=jnp.float32)
            aj_row = (jax.lax.dot_general(
                          attj_ref[...], xl, (((1,), (1,)), ((), ())),
                          preferred_element_type=jnp.float32)
                      + jax.lax.dot_general(
                          attemj_ref[...], emb, (((1,), (1,)), ((), ())),
                          preferred_element_type=jnp.float32))
            a_i = (jnp.sum(xl * atti_ref[...], axis=1, keepdims=True)
                   + jnp.sum(emb * attemi_ref[...], axis=1, keepdims=True))
            s = a_i + aj_row
            s = jnp.maximum(s, 0.2 * s)
            e = jnp.exp(s + bias_scr[...])
            denom = jnp.sum(e, axis=1, keepdims=True) + 1e-16
            u = jax.lax.dot_general(
                e, xl, (((1,), (0,)), ((), ())),
                preferred_element_type=jnp.float32)
            outv = u / denom + gb_ref[...]
            out_ref[k] = outv
            ov = jnp.where(valid, outv, 0.0)
            s1_ref[k, 0] = jnp.sum(ov, axis=0)
            s2_ref[k, 0] = jnp.sum(jnp.where(valid, outv * outv, 0.0),
                                   axis=0)


def _bn1_kernel(s1_ref, s2_ref, g1_ref, b1_ref, out_ref, emb_ref,
                sc_ref, sh_ref, t1_ref, t2_ref, st_scr):
    i = pl.program_id(0)

    @pl.when(i == 0)
    def _stats_phase():
        mu = jnp.sum(s1_ref[...], axis=(0, 1)) / _CNT
        var = jnp.sum(s2_ref[...], axis=(0, 1)) / _CNT - mu * mu
        scale = jax.lax.rsqrt(var + _EPS) * g1_ref[0]
        shift = b1_ref[0] - mu * scale
        st_scr[0, :] = scale
        st_scr[1, :] = shift
        sc_ref[0, :] = scale
        sh_ref[0, :] = shift

    @pl.when(i > 0)
    def _t_phase():
        emb = emb_ref[...]
        rows = jax.lax.broadcasted_iota(jnp.int32, (_NP, 1), 0)
        valid = rows < _N
        for k in range(_GB):
            h = out_ref[k] * st_scr[0, :][None, :] + st_scr[1, :][None, :]
            h = jnp.maximum(h, 0.0)
            xo = h * emb
            xv = jnp.where(valid, xo, 0.0)
            t1_ref[k, 0] = jnp.sum(xv, axis=0)
            t2_ref[k, 0] = jnp.sum(jnp.where(valid, xo * xo, 0.0), axis=0)


def _bn2_kernel(t1_ref, t2_ref, g2_ref, b2_ref, sc1_ref, sh1_ref, out_ref,
                emb_ref, ow_ref, ob_ref, pred_ref, st_scr):
    i = pl.program_id(0)

    @pl.when(i == 0)
    def _stats_phase():
        mu = jnp.sum(t1_ref[...], axis=(0, 1)) / _CNT
        var = jnp.sum(t2_ref[...], axis=(0, 1)) / _CNT - mu * mu
        scale = jax.lax.rsqrt(var + _EPS) * g2_ref[0]
        st_scr[0, :] = scale
        st_scr[1, :] = b2_ref[0] - mu * scale

    @pl.when(i > 0)
    def _apply_phase():
        emb = emb_ref[...]
        for k in range(_GB):
            h = out_ref[k] * sc1_ref[...] + sh1_ref[...]
            h = jnp.maximum(h, 0.0)
            xo = h * emb
            y = xo * st_scr[0, :][None, :] + st_scr[1, :][None, :]
            y = jnp.maximum(y, 0.0)
            p = jax.lax.dot_general(
                ow_ref[...], y, (((1,), (1,)), ((), ())),
                preferred_element_type=jnp.float32)
            pred_ref[k] = p + ob_ref[0, 0]


def kernel(data, org_edge_index, embedding_weight, lin_W, att_i, att_j,
           att_em_i, att_em_j, gnn_bias, bn1_gamma, bn1_beta, bn2_gamma,
           bn2_beta, out_W, out_b):
    del org_edge_index
    f32 = jnp.float32
    emb_p = jnp.zeros((_NP, _D), f32).at[:_N].set(embedding_weight)
    data_p = jnp.zeros((_B, _NP, _IN), f32).at[:, :_N].set(data)

    vec = lambda: pl.BlockSpec((1, _D), lambda i: (0, 0))
    out, s1, s2 = pl.pallas_call(
        _attn_kernel,
        grid=(_NB + _B // _AB,),
        in_specs=[
            pl.BlockSpec((_NP, _D), lambda i: (0, 0)),
            pl.BlockSpec((_BLK, _D), lambda i: (jnp.minimum(i, _NB - 1), 0)),
            pl.BlockSpec((_AB, _NP, _IN),
                         lambda i: (jnp.maximum(i - _NB, 0), 0, 0)),
            pl.BlockSpec((_D, _IN), lambda i: (0, 0)),
            vec(), vec(), vec(), vec(), vec(),
        ],
        out_specs=[
            pl.BlockSpec((_AB, _NP, _D), lambda i: (jnp.maximum(i - _NB, 0),
                                                    0, 0)),
            pl.BlockSpec((_AB, 1, _D), lambda i: (jnp.maximum(i - _NB, 0),
                                                  0, 0)),
            pl.BlockSpec((_AB, 1, _D), lambda i: (jnp.maximum(i - _NB, 0),
                                                  0, 0)),
        ],
        out_shape=[
            jax.ShapeDtypeStruct((_B, _NP, _D), f32),
            jax.ShapeDtypeStruct((_B, 1, _D), f32),
            jax.ShapeDtypeStruct((_B, 1, _D), f32),
        ],
        scratch_shapes=[pltpu.VMEM((_NP, _NP), f32)],
    )(emb_p, emb_p, data_p, lin_W, att_i.reshape(1, _D), att_j.reshape(1, _D),
      att_em_i.reshape(1, _D), att_em_j.reshape(1, _D),
      gnn_bias.reshape(1, _D))

    sc1, sh1, t1, t2 = pl.pallas_call(
        _bn1_kernel,
        grid=(1 + _B // _GB,),
        in_specs=[
            pl.BlockSpec((_B, 1, _D), lambda i: (0, 0, 0)),
            pl.BlockSpec((_B, 1, _D), lambda i: (0, 0, 0)),
            pl.BlockSpec((1, _D), lambda i: (0, 0)),
            pl.BlockSpec((1, _D), lambda i: (0, 0)),
            pl.BlockSpec((_GB, _NP, _D), lambda i: (jnp.maximum(i - 1, 0),
                                                    0, 0)),
            pl.BlockSpec((_NP, _D), lambda i: (0, 0)),
        ],
        out_specs=[
            pl.BlockSpec((1, _D), lambda i: (0, 0)),
            pl.BlockSpec((1, _D), lambda i: (0, 0)),
            pl.BlockSpec((_GB, 1, _D), lambda i: (jnp.maximum(i - 1, 0),
                                                  0, 0)),
            pl.BlockSpec((_GB, 1, _D), lambda i: (jnp.maximum(i - 1, 0),
                                                  0, 0)),
        ],
        out_shape=[
            jax.ShapeDtypeStruct((1, _D), f32),
            jax.ShapeDtypeStruct((1, _D), f32),
            jax.ShapeDtypeStruct((_B, 1, _D), f32),
            jax.ShapeDtypeStruct((_B, 1, _D), f32),
        ],
        scratch_shapes=[pltpu.VMEM((2, _D), f32)],
    )(s1, s2, bn1_gamma.reshape(1, _D), bn1_beta.reshape(1, _D), out, emb_p)

    pred = pl.pallas_call(
        _bn2_kernel,
        grid=(1 + _B // _GB,),
        in_specs=[
            pl.BlockSpec((_B, 1, _D), lambda i: (0, 0, 0)),
            pl.BlockSpec((_B, 1, _D), lambda i: (0, 0, 0)),
            pl.BlockSpec((1, _D), lambda i: (0, 0)),
            pl.BlockSpec((1, _D), lambda i: (0, 0)),
            pl.BlockSpec((1, _D), lambda i: (0, 0)),
            pl.BlockSpec((1, _D), lambda i: (0, 0)),
            pl.BlockSpec((_GB, _NP, _D), lambda i: (jnp.maximum(i - 1, 0),
                                                    0, 0)),
            pl.BlockSpec((_NP, _D), lambda i: (0, 0)),
            pl.BlockSpec((1, _D), lambda i: (0, 0)),
            pl.BlockSpec((1, 1), lambda i: (0, 0)),
        ],
        out_specs=pl.BlockSpec((_GB, 1, _NP),
                               lambda i: (jnp.maximum(i - 1, 0), 0, 0)),
        out_shape=jax.ShapeDtypeStruct((_B, 1, _NP), f32),
        scratch_shapes=[pltpu.VMEM((2, _D), f32)],
    )(t1, t2, bn2_gamma.reshape(1, _D), bn2_beta.reshape(1, _D), sc1, sh1,
      out, emb_p, out_W.reshape(1, _D), out_b.reshape(1, 1))

    return pred.reshape(_B, _NP)[:, :_N]


# speedup vs baseline: 159.5151x; 1.0531x over previous
import jax
import jax.numpy as jnp
from jax.experimental import pallas as pl
from jax.experimental.pallas import tpu as pltpu

_N = 1000
_NP = 1024
_D = 64
_IN = 16
_K = 20
_B = 32
_BLK = 128
_NB = _NP // _BLK
_AB = 4
_GB = 8
_CNT = float(_B * _N)
_EPS = 1e-5


def _attn_kernel(emb_ref, embb_ref, data_ref, lw_ref, atti_ref, attj_ref,
                 attemi_ref, attemj_ref, gb_ref, out_ref, s1_ref, s2_ref,
                 bias_scr):
    i = pl.program_id(0)

    @pl.when(i < _NB)
    def _mask_phase():
        w = emb_ref[...]
        wb = embb_ref[...]
        n_full = jnp.sqrt(jnp.maximum(jnp.sum(w * w, axis=1), 1e-12))
        n_blk = jnp.sqrt(jnp.maximum(jnp.sum(wb * wb, axis=1), 1e-12))
        cos = jax.lax.dot_general(
            wb, w, (((1,), (1,)), ((), ())),
            preferred_element_type=jnp.float32)
        cos = cos / (n_blk[:, None] * n_full[None, :])
        col = jax.lax.broadcasted_iota(jnp.int32, (_BLK, _NP), 1)
        rowg = i * _BLK + jax.lax.broadcasted_iota(jnp.int32, (_BLK, _NP), 0)
        cmask = jnp.where(col < _N, cos, -1e9)
        cur = cmask
        v_k = None
        for _ in range(_K):
            v_k = jnp.max(cur, axis=1, keepdims=True)
            cur = jnp.where(cur == v_k, -2e9, cur)
        selected = jnp.logical_or(cmask >= v_k, col == rowg)
        bias_scr[pl.ds(i * _BLK, _BLK), :] = jnp.where(
            selected, 0.0, -1e30).astype(jnp.float32)

    @pl.when(i >= _NB)
    def _attn_phase():
        emb = emb_ref[...]
        rows = jax.lax.broadcasted_iota(jnp.int32, (_NP, 1), 0)
        valid = rows < _N
        for k in range(_AB):
            db = data_ref[k]
            xl = jax.lax.dot_general(
                db, lw_ref[...], (((1,), (1,)), ((), ())),
                preferred_element_type=jnp.float32)
            aj_row = (jax.lax.dot_general(
                          attj_ref[...], xl, (((1,), (1,)), ((), ())),
                          preferred_element_type=jnp.float32)
                      + jax.lax.dot_general(
                          attemj_ref[...], emb, (((1,), (1,)), ((), ())),
                          preferred_element_type=jnp.float32))
            a_i = (jnp.sum(xl * atti_ref[...], axis=1, keepdims=True)
                   + jnp.sum(emb * attemi_ref[...], axis=1, keepdims=True))
            s = a_i + aj_row
            s = jnp.maximum(s, 0.2 * s)
            e = jnp.exp(s + bias_scr[...])
            denom = jnp.sum(e, axis=1, keepdims=True) + 1e-16
            u = jax.lax.dot_general(
                e, xl, (((1,), (0,)), ((), ())),
                preferred_element_type=jnp.float32)
            outv = u / denom + gb_ref[...]
            out_ref[k] = outv
            ov = jnp.where(valid, outv, 0.0)
            s1_ref[k, 0] = jnp.sum(ov, axis=0)
            s2_ref[k, 0] = jnp.sum(jnp.where(valid, outv * outv, 0.0),
                                   axis=0)


def _bn1_kernel(s1_ref, s2_ref, g1_ref, b1_ref, out_ref, emb_ref,
                sc_ref, sh_ref, t1_ref, t2_ref, st_scr):
    i = pl.program_id(0)

    @pl.when(i == 0)
    def _stats_phase():
        mu = jnp.sum(s1_ref[...], axis=(0, 1)) / _CNT
        var = jnp.sum(s2_ref[...], axis=(0, 1)) / _CNT - mu * mu
        scale = jax.lax.rsqrt(var + _EPS) * g1_ref[0]
        shift = b1_ref[0] - mu * scale
        st_scr[0, :] = scale
        st_scr[1, :] = shift
        sc_ref[0, :] = scale
        sh_ref[0, :] = shift

    @pl.when(i > 0)
    def _t_phase():
        emb = emb_ref[...]
        rows = jax.lax.broadcasted_iota(jnp.int32, (_NP, 1), 0)
        valid = rows < _N
        for k in range(_GB):
            h = out_ref[k] * st_scr[0, :][None, :] + st_scr[1, :][None, :]
            h = jnp.maximum(h, 0.0)
            xo = h * emb
            xv = jnp.where(valid, xo, 0.0)
            t1_ref[k, 0] = jnp.sum(xv, axis=0)
            t2_ref[k, 0] = jnp.sum(jnp.where(valid, xo * xo, 0.0), axis=0)


def _bn2_kernel(t1_ref, t2_ref, g2_ref, b2_ref, sc1_ref, sh1_ref, out_ref,
                emb_ref, ow_ref, ob_ref, pred_ref, st_scr):
    i = pl.program_id(0)

    @pl.when(i == 0)
    def _stats_phase():
        mu = jnp.sum(t1_ref[...], axis=(0, 1)) / _CNT
        var = jnp.sum(t2_ref[...], axis=(0, 1)) / _CNT - mu * mu
        scale = jax.lax.rsqrt(var + _EPS) * g2_ref[0]
        st_scr[0, :] = scale
        st_scr[1, :] = b2_ref[0] - mu * scale

    @pl.when(i > 0)
    def _apply_phase():
        emb = emb_ref[...]
        for k in range(_GB):
            h = out_ref[k] * sc1_ref[...] + sh1_ref[...]
            h = jnp.maximum(h, 0.0)
            xo = h * emb
            y = xo * st_scr[0, :][None, :] + st_scr[1, :][None, :]
            y = jnp.maximum(y, 0.0)
            p = jax.lax.dot_general(
                ow_ref[...], y, (((1,), (1,)), ((), ())),
                preferred_element_type=jnp.float32)
            pred_ref[k] = p + ob_ref[0, 0]


def kernel(data, org_edge_index, embedding_weight, lin_W, att_i, att_j,
           att_em_i, att_em_j, gnn_bias, bn1_gamma, bn1_beta, bn2_gamma,
           bn2_beta, out_W, out_b):
    del org_edge_index
    f32 = jnp.float32
    emb_p = jnp.zeros((_NP, _D), f32).at[:_N].set(embedding_weight)
    data_p = jnp.zeros((_B, _NP, _IN), f32).at[:, :_N].set(data)

    vec = lambda: pl.BlockSpec((1, _D), lambda i: (0, 0))
    out, s1, s2 = pl.pallas_call(
        _attn_kernel,
        grid=(_NB + _B // _AB,),
        in_specs=[
            pl.BlockSpec((_NP, _D), lambda i: (0, 0)),
            pl.BlockSpec((_BLK, _D), lambda i: (jnp.minimum(i, _NB - 1), 0)),
            pl.BlockSpec((_AB, _NP, _IN),
                         lambda i: (jnp.maximum(i - _NB, 0), 0, 0)),
            pl.BlockSpec((_D, _IN), lambda i: (0, 0)),
            vec(), vec(), vec(), vec(), vec(),
        ],
        out_specs=[
            pl.BlockSpec((_AB, _NP, _D), lambda i: (jnp.maximum(i - _NB, 0),
                                                    0, 0)),
            pl.BlockSpec((_AB, 1, _D), lambda i: (jnp.maximum(i - _NB, 0),
                                                  0, 0)),
            pl.BlockSpec((_AB, 1, _D), lambda i: (jnp.maximum(i - _NB, 0),
                                                  0, 0)),
        ],
        out_shape=[
            jax.ShapeDtypeStruct((_B, _NP, _D), f32),
            jax.ShapeDtypeStruct((_B, 1, _D), f32),
            jax.ShapeDtypeStruct((_B, 1, _D), f32),
        ],
        scratch_shapes=[pltpu.VMEM((_NP, _NP), f32)],
    )(emb_p, emb_p, data_p, lin_W, att_i.reshape(1, _D), att_j.reshape(1, _D),
      att_em_i.reshape(1, _D), att_em_j.reshape(1, _D),
      gnn_bias.reshape(1, _D))

    sc1, sh1, t1, t2 = pl.pallas_call(
        _bn1_kernel,
        grid=(1 + _B // _GB,),
        in_specs=[
            pl.BlockSpec((_B, 1, _D), lambda i: (0, 0, 0)),
            pl.BlockSpec((_B, 1, _D), lambda i: (0, 0, 0)),
            pl.BlockSpec((1, _D), lambda i: (0, 0)),
            pl.BlockSpec((1, _D), lambda i: (0, 0)),
            pl.BlockSpec((_GB, _NP, _D), lambda i: (jnp.maximum(i - 1, 0),
                                                    0, 0)),
            pl.BlockSpec((_NP, _D), lambda i: (0, 0)),
        ],
        out_specs=[
            pl.BlockSpec((1, _D), lambda i: (0, 0)),
            pl.BlockSpec((1, _D), lambda i: (0, 0)),
            pl.BlockSpec((_GB, 1, _D), lambda i: (jnp.maximum(i - 1, 0),
                                                  0, 0)),
            pl.BlockSpec((_GB, 1, _D), lambda i: (jnp.maximum(i - 1, 0),
                                                  0, 0)),
        ],
        out_shape=[
            jax.ShapeDtypeStruct((1, _D), f32),
            jax.ShapeDtypeStruct((1, _D), f32),
            jax.ShapeDtypeStruct((_B, 1, _D), f32),
            jax.ShapeDtypeStruct((_B, 1, _D), f32),
        ],
        scratch_shapes=[pltpu.VMEM((2, _D), f32)],
    )(s1, s2, bn1_gamma.reshape(1, _D), bn1_beta.reshape(1, _D), out, emb_p)

    pred = pl.pallas_call(
        _bn2_kernel,
        grid=(1 + _B // _GB,),
        in_specs=[
            pl.BlockSpec((_B, 1, _D), lambda i: (0, 0, 0)),
            pl.BlockSpec((_B, 1, _D), lambda i: (0, 0, 0)),
            pl.BlockSpec((1, _D), lambda i: (0, 0)),
            pl.BlockSpec((1, _D), lambda i: (0, 0)),
            pl.BlockSpec((1, _D), lambda i: (0, 0)),
            pl.BlockSpec((1, _D), lambda i: (0, 0)),
            pl.BlockSpec((_GB, _NP, _D), lambda i: (jnp.maximum(i - 1, 0),
                                                    0, 0)),
            pl.BlockSpec((_NP, _D), lambda i: (0, 0)),
            pl.BlockSpec((1, _D), lambda i: (0, 0)),
            pl.BlockSpec((1, 1), lambda i: (0, 0)),
        ],
        out_specs=pl.BlockSpec((_GB, 1, _NP),
                               lambda i: (jnp.maximum(i - 1, 0), 0, 0)),
        out_shape=jax.ShapeDtypeStruct((_B, 1, _NP), f32),
        scratch_shapes=[pltpu.VMEM((2, _D), f32)],
    )(t1, t2, bn2_gamma.reshape(1, _D), bn2_beta.reshape(1, _D), sc1, sh1,
      out, emb_p, out_W.reshape(1, _D), out_b.reshape(1, 1))

    return pred.reshape(_B, _NP)[:, :_N]


# speedup vs baseline: 161.2557x; 1.0109x over previous
import jax
import jax.numpy as jnp
from jax.experimental import pallas as pl
from jax.experimental.pallas import tpu as pltpu

_N = 1000
_NP = 1024
_D = 64
_IN = 16
_K = 20
_B = 32
_BLK = 128
_NB = _NP // _BLK
_AB = 4
_GB = 8
_CNT = float(_B * _N)
_EPS = 1e-5


def _attn_kernel(emb_ref, embb_ref, data_ref, lw_ref, atti_ref, attj_ref,
                 attemi_ref, attemj_ref, gb_ref, out_ref, s1_ref, s2_ref,
                 bias_scr):
    i = pl.program_id(0)

    @pl.when(i < _NB)
    def _mask_phase():
        w = emb_ref[...]
        wb = embb_ref[...]
        n_full = jnp.sqrt(jnp.maximum(jnp.sum(w * w, axis=1), 1e-12))
        n_blk = jnp.sqrt(jnp.maximum(jnp.sum(wb * wb, axis=1), 1e-12))
        cos = jax.lax.dot_general(
            wb, w, (((1,), (1,)), ((), ())),
            preferred_element_type=jnp.float32)
        cos = cos / (n_blk[:, None] * n_full[None, :])
        col = jax.lax.broadcasted_iota(jnp.int32, (_BLK, _NP), 1)
        rowg = i * _BLK + jax.lax.broadcasted_iota(jnp.int32, (_BLK, _NP), 0)
        cmask = jnp.where(col < _N, cos, -1e9)
        cur = cmask
        v_k = None
        for _ in range(_K):
            v_k = jnp.max(cur, axis=1, keepdims=True)
            cur = jnp.where(cur == v_k, -2e9, cur)
        selected = jnp.logical_or(cmask >= v_k, col == rowg)
        bias_scr[pl.ds(i * _BLK, _BLK), :] = jnp.where(
            selected, 0.0, -1e30).astype(jnp.float32)

    @pl.when(i >= _NB)
    def _attn_phase():
        emb = emb_ref[...]
        rows = jax.lax.broadcasted_iota(jnp.int32, (_NP, 1), 0)
        valid = rows < _N
        ones_col = jnp.ones((_NP, 1), jnp.float32)
        for k in range(_AB):
            db = data_ref[k]
            xl = jax.lax.dot_general(
                db, lw_ref[...], (((1,), (1,)), ((), ())),
                preferred_element_type=jnp.float32)
            aj_row = (jax.lax.dot_general(
                          attj_ref[...], xl, (((1,), (1,)), ((), ())),
                          preferred_element_type=jnp.float32)
                      + jax.lax.dot_general(
                          attemj_ref[...], emb, (((1,), (1,)), ((), ())),
                          preferred_element_type=jnp.float32))
            a_i = (jax.lax.dot_general(
                       xl, atti_ref[...], (((1,), (1,)), ((), ())),
                       preferred_element_type=jnp.float32)
                   + jax.lax.dot_general(
                       emb, attemi_ref[...], (((1,), (1,)), ((), ())),
                       preferred_element_type=jnp.float32))
            s = a_i + aj_row
            s = jnp.maximum(s, 0.2 * s)
            e = jnp.exp(s + bias_scr[...])
            xl_aug = jnp.concatenate([xl, ones_col], axis=1)
            u = jax.lax.dot_general(
                e, xl_aug, (((1,), (0,)), ((), ())),
                preferred_element_type=jnp.float32)
            inv = 1.0 / (u[:, _D:_D + 1] + 1e-16)
            outv = u[:, :_D] * inv + gb_ref[...]
            out_ref[k] = outv
            ov = jnp.where(valid, outv, 0.0)
            s1_ref[k, 0] = jnp.sum(ov, axis=0)
            s2_ref[k, 0] = jnp.sum(jnp.where(valid, outv * outv, 0.0),
                                   axis=0)


def _bn1_kernel(s1_ref, s2_ref, g1_ref, b1_ref, out_ref, emb_ref,
                sc_ref, sh_ref, t1_ref, t2_ref, st_scr):
    i = pl.program_id(0)

    @pl.when(i == 0)
    def _stats_phase():
        mu = jnp.sum(s1_ref[...], axis=(0, 1)) / _CNT
        var = jnp.sum(s2_ref[...], axis=(0, 1)) / _CNT - mu * mu
        scale = jax.lax.rsqrt(var + _EPS) * g1_ref[0]
        shift = b1_ref[0] - mu * scale
        st_scr[0, :] = scale
        st_scr[1, :] = shift
        sc_ref[0, :] = scale
        sh_ref[0, :] = shift

    @pl.when(i > 0)
    def _t_phase():
        emb = emb_ref[...]
        rows = jax.lax.broadcasted_iota(jnp.int32, (_NP, 1), 0)
        valid = rows < _N
        for k in range(_GB):
            h = out_ref[k] * st_scr[0, :][None, :] + st_scr[1, :][None, :]
            h = jnp.maximum(h, 0.0)
            xo = h * emb
            xv = jnp.where(valid, xo, 0.0)
            t1_ref[k, 0] = jnp.sum(xv, axis=0)
            t2_ref[k, 0] = jnp.sum(jnp.where(valid, xo * xo, 0.0), axis=0)


def _bn2_kernel(t1_ref, t2_ref, g2_ref, b2_ref, sc1_ref, sh1_ref, out_ref,
                emb_ref, ow_ref, ob_ref, pred_ref, st_scr):
    i = pl.program_id(0)

    @pl.when(i == 0)
    def _stats_phase():
        mu = jnp.sum(t1_ref[...], axis=(0, 1)) / _CNT
        var = jnp.sum(t2_ref[...], axis=(0, 1)) / _CNT - mu * mu
        scale = jax.lax.rsqrt(var + _EPS) * g2_ref[0]
        st_scr[0, :] = scale
        st_scr[1, :] = b2_ref[0] - mu * scale

    @pl.when(i > 0)
    def _apply_phase():
        emb = emb_ref[...]
        for k in range(_GB):
            h = out_ref[k] * sc1_ref[...] + sh1_ref[...]
            h = jnp.maximum(h, 0.0)
            xo = h * emb
            y = xo * st_scr[0, :][None, :] + st_scr[1, :][None, :]
            y = jnp.maximum(y, 0.0)
            p = jax.lax.dot_general(
                ow_ref[...], y, (((1,), (1,)), ((), ())),
                preferred_element_type=jnp.float32)
            pred_ref[k] = p + ob_ref[0, 0]


def kernel(data, org_edge_index, embedding_weight, lin_W, att_i, att_j,
           att_em_i, att_em_j, gnn_bias, bn1_gamma, bn1_beta, bn2_gamma,
           bn2_beta, out_W, out_b):
    del org_edge_index
    f32 = jnp.float32
    emb_p = jnp.zeros((_NP, _D), f32).at[:_N].set(embedding_weight)
    data_p = jnp.zeros((_B, _NP, _IN), f32).at[:, :_N].set(data)

    vec = lambda: pl.BlockSpec((1, _D), lambda i: (0, 0))
    out, s1, s2 = pl.pallas_call(
        _attn_kernel,
        grid=(_NB + _B // _AB,),
        in_specs=[
            pl.BlockSpec((_NP, _D), lambda i: (0, 0)),
            pl.BlockSpec((_BLK, _D), lambda i: (jnp.minimum(i, _NB - 1), 0)),
            pl.BlockSpec((_AB, _NP, _IN),
                         lambda i: (jnp.maximum(i - _NB, 0), 0, 0)),
            pl.BlockSpec((_D, _IN), lambda i: (0, 0)),
            vec(), vec(), vec(), vec(), vec(),
        ],
        out_specs=[
            pl.BlockSpec((_AB, _NP, _D), lambda i: (jnp.maximum(i - _NB, 0),
                                                    0, 0)),
            pl.BlockSpec((_AB, 1, _D), lambda i: (jnp.maximum(i - _NB, 0),
                                                  0, 0)),
            pl.BlockSpec((_AB, 1, _D), lambda i: (jnp.maximum(i - _NB, 0),
                                                  0, 0)),
        ],
        out_shape=[
            jax.ShapeDtypeStruct((_B, _NP, _D), f32),
            jax.ShapeDtypeStruct((_B, 1, _D), f32),
            jax.ShapeDtypeStruct((_B, 1, _D), f32),
        ],
        scratch_shapes=[pltpu.VMEM((_NP, _NP), f32)],
    )(emb_p, emb_p, data_p, lin_W, att_i.reshape(1, _D), att_j.reshape(1, _D),
      att_em_i.reshape(1, _D), att_em_j.reshape(1, _D),
      gnn_bias.reshape(1, _D))

    sc1, sh1, t1, t2 = pl.pallas_call(
        _bn1_kernel,
        grid=(1 + _B // _GB,),
        in_specs=[
            pl.BlockSpec((_B, 1, _D), lambda i: (0, 0, 0)),
            pl.BlockSpec((_B, 1, _D), lambda i: (0, 0, 0)),
            pl.BlockSpec((1, _D), lambda i: (0, 0)),
            pl.BlockSpec((1, _D), lambda i: (0, 0)),
            pl.BlockSpec((_GB, _NP, _D), lambda i: (jnp.maximum(i - 1, 0),
                                                    0, 0)),
            pl.BlockSpec((_NP, _D), lambda i: (0, 0)),
        ],
        out_specs=[
            pl.BlockSpec((1, _D), lambda i: (0, 0)),
            pl.BlockSpec((1, _D), lambda i: (0, 0)),
            pl.BlockSpec((_GB, 1, _D), lambda i: (jnp.maximum(i - 1, 0),
                                                  0, 0)),
            pl.BlockSpec((_GB, 1, _D), lambda i: (jnp.maximum(i - 1, 0),
                                                  0, 0)),
        ],
        out_shape=[
            jax.ShapeDtypeStruct((1, _D), f32),
            jax.ShapeDtypeStruct((1, _D), f32),
            jax.ShapeDtypeStruct((_B, 1, _D), f32),
            jax.ShapeDtypeStruct((_B, 1, _D), f32),
        ],
        scratch_shapes=[pltpu.VMEM((2, _D), f32)],
    )(s1, s2, bn1_gamma.reshape(1, _D), bn1_beta.reshape(1, _D), out, emb_p)

    pred = pl.pallas_call(
        _bn2_kernel,
        grid=(1 + _B // _GB,),
        in_specs=[
            pl.BlockSpec((_B, 1, _D), lambda i: (0, 0, 0)),
            pl.BlockSpec((_B, 1, _D), lambda i: (0, 0, 0)),
            pl.BlockSpec((1, _D), lambda i: (0, 0)),
            pl.BlockSpec((1, _D), lambda i: (0, 0)),
            pl.BlockSpec((1, _D), lambda i: (0, 0)),
            pl.BlockSpec((1, _D), lambda i: (0, 0)),
            pl.BlockSpec((_GB, _NP, _D), lambda i: (jnp.maximum(i - 1, 0),
                                                    0, 0)),
            pl.BlockSpec((_NP, _D), lambda i: (0, 0)),
            pl.BlockSpec((1, _D), lambda i: (0, 0)),
            pl.BlockSpec((1, 1), lambda i: (0, 0)),
        ],
        out_specs=pl.BlockSpec((_GB, 1, _NP),
                               lambda i: (jnp.maximum(i - 1, 0), 0, 0)),
        out_shape=jax.ShapeDtypeStruct((_B, 1, _NP), f32),
        scratch_shapes=[pltpu.VMEM((2, _D), f32)],
    )(t1, t2, bn2_gamma.reshape(1, _D), bn2_beta.reshape(1, _D), sc1, sh1,
      out, emb_p, out_W.reshape(1, _D), out_b.reshape(1, 1))

    return pred.reshape(_B, _NP)[:, :_N]


# speedup vs baseline: 180.0687x; 1.1167x over previous
import jax
import jax.numpy as jnp
from jax.experimental import pallas as pl
from jax.experimental.pallas import tpu as pltpu

_N = 1000
_NP = 1024
_D = 64
_IN = 16
_K = 20
_B = 32
_BLK = 128
_NB = _NP // _BLK
_AB = 4
_GB = 8
_CNT = float(_B * _N)
_EPS = 1e-5


def _attn_kernel(emb_ref, embb_ref, data_ref, lw_ref, atti_ref, attj_ref,
                 attemi_ref, attemj_ref, gb_ref, out_ref, s1_ref, s2_ref,
                 bias_scr):
    i = pl.program_id(0)

    @pl.when(i < _NB)
    def _mask_phase():
        w = emb_ref[...]
        wb = embb_ref[...]
        n_full = jnp.sqrt(jnp.maximum(jnp.sum(w * w, axis=1), 1e-12))
        n_blk = jnp.sqrt(jnp.maximum(jnp.sum(wb * wb, axis=1), 1e-12))
        cos = jax.lax.dot_general(
            wb, w, (((1,), (1,)), ((), ())),
            preferred_element_type=jnp.float32)
        cos = cos / (n_blk[:, None] * n_full[None, :])
        col = jax.lax.broadcasted_iota(jnp.int32, (_BLK, _NP), 1)
        rowg = i * _BLK + jax.lax.broadcasted_iota(jnp.int32, (_BLK, _NP), 0)
        cmask = jnp.where(col < _N, cos, -1e9)
        cur = cmask
        v_k = None
        for _ in range(_K):
            v_k = jnp.max(cur, axis=1, keepdims=True)
            cur = jnp.where(cur == v_k, -2e9, cur)
        selected = jnp.logical_or(cmask >= v_k, col == rowg)
        bias_scr[pl.ds(i * _BLK, _BLK), :] = jnp.where(
            selected, 1.0, 0.0).astype(jnp.float32)

    @pl.when(i >= _NB)
    def _attn_phase():
        emb = emb_ref[...]
        rows = jax.lax.broadcasted_iota(jnp.int32, (_NP, 1), 0)
        valid = rows < _N
        ones_col = jnp.ones((_NP, 1), jnp.float32)
        for k in range(_AB):
            db = data_ref[k]
            xl = jax.lax.dot_general(
                db, lw_ref[...], (((1,), (1,)), ((), ())),
                preferred_element_type=jnp.float32)
            aj_row = (jax.lax.dot_general(
                          attj_ref[...], xl, (((1,), (1,)), ((), ())),
                          preferred_element_type=jnp.float32)
                      + jax.lax.dot_general(
                          attemj_ref[...], emb, (((1,), (1,)), ((), ())),
                          preferred_element_type=jnp.float32))
            a_i = (jax.lax.dot_general(
                       xl, atti_ref[...], (((1,), (1,)), ((), ())),
                       preferred_element_type=jnp.float32)
                   + jax.lax.dot_general(
                       emb, attemi_ref[...], (((1,), (1,)), ((), ())),
                       preferred_element_type=jnp.float32))
            epi = jnp.exp(a_i)
            eni = jnp.exp(0.2 * a_i)
            epj = jnp.exp(aj_row)
            enj = jnp.exp(0.2 * aj_row)
            e = jnp.maximum(epi * epj, eni * enj) * bias_scr[...]
            xl_aug = jnp.concatenate([xl, ones_col], axis=1)
            u = jax.lax.dot_general(
                e, xl_aug, (((1,), (0,)), ((), ())),
                preferred_element_type=jnp.float32)
            inv = 1.0 / (u[:, _D:_D + 1] + 1e-16)
            outv = u[:, :_D] * inv + gb_ref[...]
            out_ref[k] = outv
            ov = jnp.where(valid, outv, 0.0)
            s1_ref[k, 0] = jnp.sum(ov, axis=0)
            s2_ref[k, 0] = jnp.sum(jnp.where(valid, outv * outv, 0.0),
                                   axis=0)


def _bn1_kernel(s1_ref, s2_ref, g1_ref, b1_ref, out_ref, emb_ref,
                sc_ref, sh_ref, t1_ref, t2_ref, st_scr):
    i = pl.program_id(0)

    @pl.when(i == 0)
    def _stats_phase():
        mu = jnp.sum(s1_ref[...], axis=(0, 1)) / _CNT
        var = jnp.sum(s2_ref[...], axis=(0, 1)) / _CNT - mu * mu
        scale = jax.lax.rsqrt(var + _EPS) * g1_ref[0]
        shift = b1_ref[0] - mu * scale
        st_scr[0, :] = scale
        st_scr[1, :] = shift
        sc_ref[0, :] = scale
        sh_ref[0, :] = shift

    @pl.when(i > 0)
    def _t_phase():
        emb = emb_ref[...]
        rows = jax.lax.broadcasted_iota(jnp.int32, (_NP, 1), 0)
        valid = rows < _N
        for k in range(_GB):
            h = out_ref[k] * st_scr[0, :][None, :] + st_scr[1, :][None, :]
            h = jnp.maximum(h, 0.0)
            xo = h * emb
            xv = jnp.where(valid, xo, 0.0)
            t1_ref[k, 0] = jnp.sum(xv, axis=0)
            t2_ref[k, 0] = jnp.sum(jnp.where(valid, xo * xo, 0.0), axis=0)


def _bn2_kernel(t1_ref, t2_ref, g2_ref, b2_ref, sc1_ref, sh1_ref, out_ref,
                emb_ref, ow_ref, ob_ref, pred_ref, st_scr):
    i = pl.program_id(0)

    @pl.when(i == 0)
    def _stats_phase():
        mu = jnp.sum(t1_ref[...], axis=(0, 1)) / _CNT
        var = jnp.sum(t2_ref[...], axis=(0, 1)) / _CNT - mu * mu
        scale = jax.lax.rsqrt(var + _EPS) * g2_ref[0]
        st_scr[0, :] = scale
        st_scr[1, :] = b2_ref[0] - mu * scale

    @pl.when(i > 0)
    def _apply_phase():
        emb = emb_ref[...]
        for k in range(_GB):
            h = out_ref[k] * sc1_ref[...] + sh1_ref[...]
            h = jnp.maximum(h, 0.0)
            xo = h * emb
            y = xo * st_scr[0, :][None, :] + st_scr[1, :][None, :]
            y = jnp.maximum(y, 0.0)
            p = jax.lax.dot_general(
                ow_ref[...], y, (((1,), (1,)), ((), ())),
                preferred_element_type=jnp.float32)
            pred_ref[k] = p + ob_ref[0, 0]


def kernel(data, org_edge_index, embedding_weight, lin_W, att_i, att_j,
           att_em_i, att_em_j, gnn_bias, bn1_gamma, bn1_beta, bn2_gamma,
           bn2_beta, out_W, out_b):
    del org_edge_index
    f32 = jnp.float32
    emb_p = jnp.zeros((_NP, _D), f32).at[:_N].set(embedding_weight)
    data_p = jnp.zeros((_B, _NP, _IN), f32).at[:, :_N].set(data)

    vec = lambda: pl.BlockSpec((1, _D), lambda i: (0, 0))
    out, s1, s2 = pl.pallas_call(
        _attn_kernel,
        grid=(_NB + _B // _AB,),
        in_specs=[
            pl.BlockSpec((_NP, _D), lambda i: (0, 0)),
            pl.BlockSpec((_BLK, _D), lambda i: (jnp.minimum(i, _NB - 1), 0)),
            pl.BlockSpec((_AB, _NP, _IN),
                         lambda i: (jnp.maximum(i - _NB, 0), 0, 0)),
            pl.BlockSpec((_D, _IN), lambda i: (0, 0)),
            vec(), vec(), vec(), vec(), vec(),
        ],
        out_specs=[
            pl.BlockSpec((_AB, _NP, _D), lambda i: (jnp.maximum(i - _NB, 0),
                                                    0, 0)),
            pl.BlockSpec((_AB, 1, _D), lambda i: (jnp.maximum(i - _NB, 0),
                                                  0, 0)),
            pl.BlockSpec((_AB, 1, _D), lambda i: (jnp.maximum(i - _NB, 0),
                                                  0, 0)),
        ],
        out_shape=[
            jax.ShapeDtypeStruct((_B, _NP, _D), f32),
            jax.ShapeDtypeStruct((_B, 1, _D), f32),
            jax.ShapeDtypeStruct((_B, 1, _D), f32),
        ],
        scratch_shapes=[pltpu.VMEM((_NP, _NP), f32)],
    )(emb_p, emb_p, data_p, lin_W, att_i.reshape(1, _D), att_j.reshape(1, _D),
      att_em_i.reshape(1, _D), att_em_j.reshape(1, _D),
      gnn_bias.reshape(1, _D))

    sc1, sh1, t1, t2 = pl.pallas_call(
        _bn1_kernel,
        grid=(1 + _B // _GB,),
        in_specs=[
            pl.BlockSpec((_B, 1, _D), lambda i: (0, 0, 0)),
            pl.BlockSpec((_B, 1, _D), lambda i: (0, 0, 0)),
            pl.BlockSpec((1, _D), lambda i: (0, 0)),
            pl.BlockSpec((1, _D), lambda i: (0, 0)),
            pl.BlockSpec((_GB, _NP, _D), lambda i: (jnp.maximum(i - 1, 0),
                                                    0, 0)),
            pl.BlockSpec((_NP, _D), lambda i: (0, 0)),
        ],
        out_specs=[
            pl.BlockSpec((1, _D), lambda i: (0, 0)),
            pl.BlockSpec((1, _D), lambda i: (0, 0)),
            pl.BlockSpec((_GB, 1, _D), lambda i: (jnp.maximum(i - 1, 0),
                                                  0, 0)),
            pl.BlockSpec((_GB, 1, _D), lambda i: (jnp.maximum(i - 1, 0),
                                                  0, 0)),
        ],
        out_shape=[
            jax.ShapeDtypeStruct((1, _D), f32),
            jax.ShapeDtypeStruct((1, _D), f32),
            jax.ShapeDtypeStruct((_B, 1, _D), f32),
            jax.ShapeDtypeStruct((_B, 1, _D), f32),
        ],
        scratch_shapes=[pltpu.VMEM((2, _D), f32)],
    )(s1, s2, bn1_gamma.reshape(1, _D), bn1_beta.reshape(1, _D), out, emb_p)

    pred = pl.pallas_call(
        _bn2_kernel,
        grid=(1 + _B // _GB,),
        in_specs=[
            pl.BlockSpec((_B, 1, _D), lambda i: (0, 0, 0)),
            pl.BlockSpec((_B, 1, _D), lambda i: (0, 0, 0)),
            pl.BlockSpec((1, _D), lambda i: (0, 0)),
            pl.BlockSpec((1, _D), lambda i: (0, 0)),
            pl.BlockSpec((1, _D), lambda i: (0, 0)),
            pl.BlockSpec((1, _D), lambda i: (0, 0)),
            pl.BlockSpec((_GB, _NP, _D), lambda i: (jnp.maximum(i - 1, 0),
                                                    0, 0)),
            pl.BlockSpec((_NP, _D), lambda i: (0, 0)),
            pl.BlockSpec((1, _D), lambda i: (0, 0)),
            pl.BlockSpec((1, 1), lambda i: (0, 0)),
        ],
        out_specs=pl.BlockSpec((_GB, 1, _NP),
                               lambda i: (jnp.maximum(i - 1, 0), 0, 0)),
        out_shape=jax.ShapeDtypeStruct((_B, 1, _NP), f32),
        scratch_shapes=[pltpu.VMEM((2, _D), f32)],
    )(t1, t2, bn2_gamma.reshape(1, _D), bn2_beta.reshape(1, _D), sc1, sh1,
      out, emb_p, out_W.reshape(1, _D), out_b.reshape(1, 1))

    return pred.reshape(_B, _NP)[:, :_N]
